# baseline (speedup 1.0000x reference)
import functools

import jax
import jax.numpy as jnp
from jax import lax
from jax.experimental import pallas as pl
from jax.experimental.pallas import tpu as pltpu
from jax.experimental.pallas import tpu_sc as plsc

_W = 256
_NGRP = 8
_TPG = 16


def kernel(batch, table):
    B, = batch.shape
    V, D = table.shape
    NTILE = _NGRP * _TPG
    V_PHYS = ((V + 127) // 128) * 128

    info = plsc.get_sparse_core_info()
    NC, NS = info.num_cores, info.num_subcores
    NW = NC * NS
    r_per_w = V // NW
    max_col0 = V_PHYS - _W
    assert max_col0 % 128 == 0

    mesh = plsc.VectorSubcoreMesh(core_axis_name="c", subcore_axis_name="s")

    @functools.partial(
        pl.kernel,
        mesh=mesh,
        out_type=jax.ShapeDtypeStruct((B, D), jnp.float32),
        compiler_params=pltpu.CompilerParams(needs_layout_passes=False),
        scratch_types=[
            pltpu.VMEM((B,), jnp.int32),
            pltpu.VMEM((B + 16,), jnp.int32),
            pltpu.VMEM((B,), jnp.int32),
            pltpu.VMEM((B + 16,), jnp.int32),
            pltpu.VMEM((B,), jnp.int32),
            pltpu.VMEM((16,), jnp.int32),
            pltpu.VMEM((16,), jnp.int32),
            pltpu.VMEM((16, D), jnp.float32),
            pltpu.VMEM((D,), jnp.int32),
            pltpu.VMEM((D, _W), jnp.float32),
            pltpu.VMEM((D, _W), jnp.float32),
            pltpu.SemaphoreType.DMA,
            pltpu.SemaphoreType.DMA,
            *[pltpu.SemaphoreType.DMA for _ in range(16)],
        ],
    )
    def gather_kernel(idx_hbm, table_t_hbm, out_hbm, gidx, locv, locp,
                      lv2, lp2, tv, tp, staging, dummy, slab0, slab1,
                      ssem0, ssem1, *lsems):
        slabs = (slab0, slab1)
        ssems = (ssem0, ssem1)
        wid = lax.axis_index("s") * NC + lax.axis_index("c")
        lo = wid * r_per_w
        hi = lo + r_per_w
        t_start = lo // _W

        pltpu.sync_copy(idx_hbm, gidx)
        for l in range(16):
            pltpu.make_async_copy(
                idx_hbm.at[pl.ds(0, D)], dummy, lsems[l]
            ).start()

        lane16 = lax.iota(jnp.int32, 16)

        def tile_col0(t_idx):
            return pl.multiple_of(
                jnp.minimum((t_start + t_idx) * _W, max_col0), 128
            )

        def fire(t_idx, slot):
            pltpu.make_async_copy(
                table_t_hbm.at[:, pl.ds(tile_col0(t_idx), _W)],
                slabs[slot], ssems[slot],
            ).start()

        def drain(slot):
            pltpu.make_async_copy(
                table_t_hbm.at[:, pl.ds(0, _W)], slabs[slot], ssems[slot]
            ).wait()

        fire(0, 0)

        def scan_a(c, cnt):
            vec = gidx[pl.ds(c * 16, 16)]
            mask = (vec >= lo) & (vec < hi)
            m = plsc.all_reduce_population_count(mask)[0]

            @pl.when(m > 0)
            def _():
                plsc.store_compressed(locv.at[pl.ds(cnt, 16)], vec, mask=mask)
                plsc.store_compressed(
                    locp.at[pl.ds(cnt, 16)], lane16 + c * 16, mask=mask
                )

            return cnt + m

        cnt = lax.fori_loop(0, B // 16, scan_a, 0)
        locv[pl.ds(cnt, 16)] = jnp.full((16,), -1, jnp.int32)

        def group(gi, _):
            g_lo = t_start + gi * _TPG

            def scan_b(c, cnt2):
                vec = locv[pl.ds(c * 16, 16)]
                pos = locp[pl.ds(c * 16, 16)]
                tile_of = vec // _W
                mask = (tile_of >= g_lo) & (tile_of < g_lo + _TPG)
                m = plsc.all_reduce_population_count(mask)[0]

                @pl.when(m > 0)
                def _():
                    plsc.store_compressed(
                        lv2.at[pl.ds(cnt2, 16)], vec, mask=mask
                    )
                    plsc.store_compressed(
                        lp2.at[pl.ds(cnt2, 16)], pos, mask=mask
                    )

                return cnt2 + m

            cnt2 = lax.fori_loop(0, (cnt + 15) // 16, scan_b, 0)
            lv2[pl.ds(cnt2, 16)] = jnp.full((16,), -1, jnp.int32)

            def pair(pp, _):
                for parity in range(2):
                    t_idx = gi * _TPG + pp * 2 + parity
                    slot = parity
                    drain(slot)
                    fire(t_idx + 1, 1 - parity)
                    t_g = t_start + t_idx
                    col0 = tile_col0(t_idx)

                    def scan_c(c, _, slot=slot, t_g=t_g, col0=col0):
                        vec = lv2[pl.ds(c * 16, 16)]
                        mask = (vec // _W) == t_g
                        m16 = plsc.all_reduce_population_count(mask)[0]

                        @pl.when(m16 > 0)
                        def _():
                            pos = lp2[pl.ds(c * 16, 16)]
                            plsc.store_compressed(
                                tv.at[pl.ds(0, 16)], vec, mask=mask
                            )
                            plsc.store_compressed(
                                tp.at[pl.ds(0, 16)], pos, mask=mask
                            )
                            tvv = tv[pl.ds(0, 16)]
                            tpv = tp[pl.ds(0, 16)]
                            jv = jnp.clip(tvv - col0, 0, _W - 1)
                            for l in range(16):
                                pltpu.make_async_copy(
                                    idx_hbm.at[pl.ds(0, D)], dummy, lsems[l]
                                ).wait()
                            for l in range(16):
                                j = jnp.full((16,), jv[l], jnp.int32)
                                for t4 in range(D // 16):
                                    g = plsc.load_gather(
                                        slabs[slot], [lane16 + 16 * t4, j]
                                    )
                                    staging[l, pl.ds(16 * t4, 16)] = g
                            for l in range(16):
                                @pl.when(l < m16)
                                def _(l=l):
                                    pltpu.make_async_copy(
                                        staging.at[pl.ds(l, 1)],
                                        out_hbm.at[pl.ds(tpv[l], 1)],
                                        lsems[l],
                                    ).start()
                                @pl.when(l >= m16)
                                def _(l=l):
                                    pltpu.make_async_copy(
                                        idx_hbm.at[pl.ds(0, D)], dummy, lsems[l]
                                    ).start()

                        return 0

                    lax.fori_loop(0, (cnt2 + 15) // 16, scan_c, 0)
                return 0

            lax.fori_loop(0, _TPG // 2, pair, 0)
            return 0

        lax.fori_loop(0, _NGRP, group, 0)

        drain(NTILE % 2)
        for l in range(16):
            pltpu.make_async_copy(
                idx_hbm.at[pl.ds(0, D)], dummy, lsems[l]
            ).wait()

    return gather_kernel(batch.astype(jnp.int32), table.T)


# speedup vs baseline: 8.2245x; 8.2245x over previous
import functools

import jax
import jax.numpy as jnp
from jax import lax
from jax.experimental import pallas as pl
from jax.experimental.pallas import tpu as pltpu
from jax.experimental.pallas import tpu_sc as plsc

_W = 256
_NGRP = 8
_TPG = 16


def kernel(batch, table):
    B, = batch.shape
    V, D = table.shape
    NTILE = _NGRP * _TPG
    V_PHYS = ((V + 127) // 128) * 128

    info = plsc.get_sparse_core_info()
    NC, NS = info.num_cores, info.num_subcores
    NW = NC * NS
    r_per_w = V // NW
    max_col0 = V_PHYS - _W
    assert max_col0 % 128 == 0

    mesh = plsc.VectorSubcoreMesh(core_axis_name="c", subcore_axis_name="s")

    @functools.partial(
        pl.kernel,
        mesh=mesh,
        out_type=jax.ShapeDtypeStruct((B, D), jnp.float32),
        compiler_params=pltpu.CompilerParams(needs_layout_passes=False),
        scratch_types=[
            pltpu.VMEM((B,), jnp.int32),
            pltpu.VMEM((B + 16,), jnp.int32),
            pltpu.VMEM((B,), jnp.int32),
            pltpu.VMEM((B + 16,), jnp.int32),
            pltpu.VMEM((B,), jnp.int32),
            pltpu.VMEM((16,), jnp.int32),
            pltpu.VMEM((16,), jnp.int32),
            pltpu.VMEM((32, D), jnp.float32),
            pltpu.VMEM((16 * D,), jnp.int32),
            pltpu.VMEM((D, _W), jnp.float32),
            pltpu.VMEM((D, _W), jnp.float32),
            pltpu.SemaphoreType.DMA,
            pltpu.SemaphoreType.DMA,
            pltpu.SemaphoreType.DMA,
        ],
    )
    def gather_kernel(idx_hbm, table_t_hbm, out_hbm, gidx, locv, locp,
                      lv2, lp2, tv, tp, staging, dummy, slab0, slab1,
                      ssem0, ssem1, osem):
        slabs = (slab0, slab1)
        ssems = (ssem0, ssem1)
        wid = lax.axis_index("s") * NC + lax.axis_index("c")
        lo = wid * r_per_w
        hi = lo + r_per_w
        t_start = lo // _W

        pltpu.sync_copy(idx_hbm, gidx)
        for _ in range(2):
            pltpu.make_async_copy(
                idx_hbm.at[pl.ds(0, 16 * D)], dummy, osem
            ).start()

        lane16 = lax.iota(jnp.int32, 16)

        def tile_col0(t_idx):
            return pl.multiple_of(
                jnp.minimum((t_start + t_idx) * _W, max_col0), 128
            )

        def fire(t_idx, slot):
            pltpu.make_async_copy(
                table_t_hbm.at[:, pl.ds(tile_col0(t_idx), _W)],
                slabs[slot], ssems[slot],
            ).start()

        def drain(slot):
            pltpu.make_async_copy(
                table_t_hbm.at[:, pl.ds(0, _W)], slabs[slot], ssems[slot]
            ).wait()

        fire(0, 0)

        def scan_a(c, cnt):
            vec = gidx[pl.ds(c * 16, 16)]
            mask = (vec >= lo) & (vec < hi)
            m = plsc.all_reduce_population_count(mask)[0]

            @pl.when(m > 0)
            def _():
                plsc.store_compressed(locv.at[pl.ds(cnt, 16)], vec, mask=mask)
                plsc.store_compressed(
                    locp.at[pl.ds(cnt, 16)], lane16 + c * 16, mask=mask
                )

            return cnt + m

        cnt = lax.fori_loop(0, B // 16, scan_a, 0)
        locv[pl.ds(cnt, 16)] = jnp.full((16,), -1, jnp.int32)

        def group(gi, gc_grp):
            g_lo = t_start + gi * _TPG

            def scan_b(c, cnt2):
                vec = locv[pl.ds(c * 16, 16)]
                pos = locp[pl.ds(c * 16, 16)]
                tile_of = vec // _W
                mask = (tile_of >= g_lo) & (tile_of < g_lo + _TPG)
                m = plsc.all_reduce_population_count(mask)[0]

                @pl.when(m > 0)
                def _():
                    plsc.store_compressed(
                        lv2.at[pl.ds(cnt2, 16)], vec, mask=mask
                    )
                    plsc.store_compressed(
                        lp2.at[pl.ds(cnt2, 16)], pos, mask=mask
                    )

                return cnt2 + m

            cnt2 = lax.fori_loop(0, (cnt + 15) // 16, scan_b, 0)
            lv2[pl.ds(cnt2, 16)] = jnp.full((16,), -1, jnp.int32)

            def pair(pp, gc_in):
                gc_box = [gc_in]
                for parity in range(2):
                    t_idx = gi * _TPG + pp * 2 + parity
                    slot = parity
                    drain(slot)
                    fire(t_idx + 1, 1 - parity)
                    t_g = t_start + t_idx
                    col0 = tile_col0(t_idx)

                    def scan_c(c, gc, slot=slot, t_g=t_g, col0=col0):
                        vec = lv2[pl.ds(c * 16, 16)]
                        mask = (vec // _W) == t_g
                        m16 = plsc.all_reduce_population_count(mask)[0]

                        @pl.when(m16 > 0)
                        def _():
                            pos = lp2[pl.ds(c * 16, 16)]
                            plsc.store_compressed(
                                tv.at[pl.ds(0, 16)], vec, mask=mask
                            )
                            plsc.store_compressed(
                                tp.at[pl.ds(0, 16)], pos, mask=mask
                            )
                            tvv = tv[pl.ds(0, 16)]
                            tpv = tp[pl.ds(0, 16)]
                            lvalid = lane16 < m16
                            jv = jnp.where(
                                lvalid,
                                jnp.clip(tvv - col0, 0, _W - 1),
                                jnp.full((16,), tvv[0] - col0, jnp.int32),
                            )
                            tp2 = jnp.where(
                                lvalid, tpv,
                                jnp.full((16,), tpv[0], jnp.int32),
                            )
                            base = (gc % 2) * 16
                            pltpu.make_async_copy(
                                idx_hbm.at[pl.ds(0, 16 * D)], dummy, osem
                            ).wait()
                            for l in range(16):
                                j = jnp.full((16,), jv[l], jnp.int32)
                                for t4 in range(D // 16):
                                    g = plsc.load_gather(
                                        slabs[slot], [lane16 + 16 * t4, j]
                                    )
                                    staging[base + l, pl.ds(16 * t4, 16)] = g
                            for l in range(16):
                                pltpu.make_async_copy(
                                    staging.at[pl.ds(base + l, 1)],
                                    out_hbm.at[pl.ds(tp2[l], 1)],
                                    osem,
                                ).start()

                        return gc + jnp.where(m16 > 0, 1, 0)

                    gc_box[0] = lax.fori_loop(
                        0, (cnt2 + 15) // 16, scan_c, gc_box[0]
                    )
                return gc_box[0]

            return lax.fori_loop(0, _TPG // 2, pair, gc_grp)

        lax.fori_loop(0, _NGRP, group, 0)

        drain(NTILE % 2)
        for _ in range(2):
            pltpu.make_async_copy(
                idx_hbm.at[pl.ds(0, 16 * D)], dummy, osem
            ).wait()

    return gather_kernel(batch.astype(jnp.int32), table.T)
